# baseline (speedup 1.0000x reference)
import functools

import jax
import jax.numpy as jnp
from jax.experimental import pallas as pl
from jax.experimental.pallas import tpu as pltpu

NB = 8192
EMB = 64
KAUG = 128
BATCH = 1024
PB = 1024
NSTEP = NB // PB
SCALE = 3.0


def _loss_kernel(x_ref, t_ref, p_ref, out_ref, xaug_ref, s_ref, mt_ref):
    j = pl.program_id(0)

    @pl.when(j == 0)
    def _():
        X = x_ref[...]
        sqx = jnp.sum(X * X, axis=1, keepdims=True)
        x2 = X * ((2.0 * SCALE) / jnp.sqrt(sqx + 1e-12))
        xaug_ref[...] = jnp.concatenate(
            [x2, jnp.ones((BATCH, 1), jnp.float32),
             jnp.zeros((BATCH, KAUG - EMB - 1), jnp.float32)],
            axis=1).astype(jnp.bfloat16)
        s_ref[...] = jnp.zeros_like(s_ref)
        mt_ref[...] = jnp.zeros_like(mt_ref)

    P = p_ref[...]
    sqp = jnp.sum(P * P, axis=1, keepdims=True)
    Pn = P * (SCALE / jnp.sqrt(sqp + 1e-12))
    nsqpn = -(SCALE * SCALE) * sqp / (sqp + 1e-12)
    paug = jnp.concatenate(
        [Pn, nsqpn, jnp.zeros((PB, KAUG - EMB - 1), jnp.float32)],
        axis=1).astype(jnp.bfloat16)

    m = jax.lax.dot_general(
        xaug_ref[...], paug, (((1,), (1,)), ((), ())),
        preferred_element_type=jnp.float32)

    s_ref[...] += jnp.sum(jnp.exp(m), axis=1, keepdims=True)

    t = t_ref[...]
    cols = j * PB + jax.lax.broadcasted_iota(jnp.int32, (BATCH, PB), 1)
    mt_ref[...] += jnp.sum(jnp.where(cols == t, m, 0.0), axis=1,
                           keepdims=True)

    @pl.when(j == NSTEP - 1)
    def _():
        out_ref[0, 0] = jnp.sum(jnp.log(s_ref[...]) - mt_ref[...]) * (
            1.0 / BATCH)


@functools.partial(jax.jit, static_argnames=())
def kernel(X, indices, T, proxies):
    del indices
    t2 = T.reshape(BATCH, 1)
    out = pl.pallas_call(
        _loss_kernel,
        grid=(NSTEP,),
        in_specs=[
            pl.BlockSpec((BATCH, EMB), lambda j: (0, 0)),
            pl.BlockSpec((BATCH, 1), lambda j: (0, 0)),
            pl.BlockSpec((PB, EMB), lambda j: (j, 0)),
        ],
        out_specs=pl.BlockSpec((1, 1), lambda j: (0, 0),
                               memory_space=pltpu.SMEM),
        out_shape=jax.ShapeDtypeStruct((1, 1), jnp.float32),
        scratch_shapes=[
            pltpu.VMEM((BATCH, KAUG), jnp.bfloat16),
            pltpu.VMEM((BATCH, 1), jnp.float32),
            pltpu.VMEM((BATCH, 1), jnp.float32),
        ],
        compiler_params=pltpu.CompilerParams(
            dimension_semantics=("arbitrary",)),
    )(X, t2, proxies)
    return jnp.reshape(out, ())


# speedup vs baseline: 1.0385x; 1.0385x over previous
import functools

import jax
import jax.numpy as jnp
from jax.experimental import pallas as pl
from jax.experimental.pallas import tpu as pltpu

NB = 8192
EMB = 64
KAUG = 128
BATCH = 1024
PB = 2048
NSTEP = NB // PB
SCALE = 3.0


def _loss_kernel(x_ref, t_ref, p_ref, out_ref, xaug_ref, s_ref, mt_ref):
    j = pl.program_id(0)

    @pl.when(j == 0)
    def _():
        X = x_ref[...]
        sqx = jnp.sum(X * X, axis=1, keepdims=True)
        x2 = X * ((2.0 * SCALE) / jnp.sqrt(sqx + 1e-12))
        xaug_ref[...] = jnp.concatenate(
            [x2, jnp.ones((BATCH, 1), jnp.float32),
             jnp.zeros((BATCH, KAUG - EMB - 1), jnp.float32)],
            axis=1).astype(jnp.bfloat16)
        s_ref[...] = jnp.zeros_like(s_ref)
        mt_ref[...] = jnp.zeros_like(mt_ref)

    P = p_ref[...]
    sqp = jnp.sum(P * P, axis=1, keepdims=True)
    Pn = P * (SCALE / jnp.sqrt(sqp + 1e-12))
    nsqpn = -(SCALE * SCALE) * sqp / (sqp + 1e-12)
    paug = jnp.concatenate(
        [Pn, nsqpn, jnp.zeros((PB, KAUG - EMB - 1), jnp.float32)],
        axis=1).astype(jnp.bfloat16)

    m = jax.lax.dot_general(
        xaug_ref[...], paug, (((1,), (1,)), ((), ())),
        preferred_element_type=jnp.float32)

    s_ref[...] += jnp.sum(jnp.exp(m), axis=1, keepdims=True)

    t = t_ref[...]
    cols = j * PB + jax.lax.broadcasted_iota(jnp.int32, (BATCH, PB), 1)
    mt_ref[...] += jnp.sum(jnp.where(cols == t, m, 0.0), axis=1,
                           keepdims=True)

    @pl.when(j == NSTEP - 1)
    def _():
        out_ref[0, 0] = jnp.sum(jnp.log(s_ref[...]) - mt_ref[...]) * (
            1.0 / BATCH)


@functools.partial(jax.jit, static_argnames=())
def kernel(X, indices, T, proxies):
    del indices
    t2 = T.reshape(BATCH, 1)
    out = pl.pallas_call(
        _loss_kernel,
        grid=(NSTEP,),
        in_specs=[
            pl.BlockSpec((BATCH, EMB), lambda j: (0, 0)),
            pl.BlockSpec((BATCH, 1), lambda j: (0, 0)),
            pl.BlockSpec((PB, EMB), lambda j: (j, 0)),
        ],
        out_specs=pl.BlockSpec((1, 1), lambda j: (0, 0),
                               memory_space=pltpu.SMEM),
        out_shape=jax.ShapeDtypeStruct((1, 1), jnp.float32),
        scratch_shapes=[
            pltpu.VMEM((BATCH, KAUG), jnp.bfloat16),
            pltpu.VMEM((BATCH, 1), jnp.float32),
            pltpu.VMEM((BATCH, 1), jnp.float32),
        ],
        compiler_params=pltpu.CompilerParams(
            dimension_semantics=("arbitrary",)),
    )(X, t2, proxies)
    return jnp.reshape(out, ())


# speedup vs baseline: 1.0973x; 1.0567x over previous
import functools

import jax
import jax.numpy as jnp
from jax.experimental import pallas as pl
from jax.experimental.pallas import tpu as pltpu

NB = 8192
EMB = 64
KAUG = 128
BATCH = 1024
PB = 2048
NSTEP = NB // PB
SCALE = 3.0


def _aug(V, width):
    sq = jnp.sum(V * V, axis=1, keepdims=True)
    Vn = V * (width / jnp.sqrt(sq + 1e-12))
    nsq = -(SCALE * SCALE) * sq / (sq + 1e-12)
    n = V.shape[0]
    return jnp.concatenate(
        [Vn, nsq, jnp.zeros((n, KAUG - EMB - 1), jnp.float32)],
        axis=1).astype(jnp.bfloat16)


def _loss_kernel(x_ref, t_ref, p_ref, out_ref, xaug_ref, s_ref, mt_ref):
    j = pl.program_id(0)

    @pl.when(j == 0)
    def _():
        X2 = x_ref[...]
        sqxe = jnp.sum(X2[:, :EMB] * X2[:, :EMB], axis=1, keepdims=True)
        sqxo = jnp.sum(X2[:, EMB:] * X2[:, EMB:], axis=1, keepdims=True)
        x2e = X2[:, :EMB] * ((2.0 * SCALE) / jnp.sqrt(sqxe + 1e-12))
        x2o = X2[:, EMB:] * ((2.0 * SCALE) / jnp.sqrt(sqxo + 1e-12))
        h = BATCH // 2
        ones = jnp.ones((h, 1), jnp.float32)
        zeros = jnp.zeros((h, KAUG - EMB - 1), jnp.float32)
        auge = jnp.concatenate([x2e, ones, zeros], axis=1)
        augo = jnp.concatenate([x2o, ones, zeros], axis=1)
        xaug_ref[...] = jnp.concatenate(
            [auge, augo], axis=0).astype(jnp.bfloat16)
        s_ref[...] = jnp.zeros_like(s_ref)
        mt_ref[...] = jnp.zeros_like(mt_ref)

    P2 = p_ref[...]
    sqpe = jnp.sum(P2[:, :EMB] * P2[:, :EMB], axis=1, keepdims=True)
    sqpo = jnp.sum(P2[:, EMB:] * P2[:, EMB:], axis=1, keepdims=True)
    hp = PB // 2
    pne = P2[:, :EMB] * (SCALE / jnp.sqrt(sqpe + 1e-12))
    pno = P2[:, EMB:] * (SCALE / jnp.sqrt(sqpo + 1e-12))
    nsqe = -(SCALE * SCALE) * sqpe / (sqpe + 1e-12)
    nsqo = -(SCALE * SCALE) * sqpo / (sqpo + 1e-12)
    pz = jnp.zeros((hp, KAUG - EMB - 1), jnp.float32)
    paug = jnp.concatenate(
        [jnp.concatenate([pne, nsqe, pz], axis=1),
         jnp.concatenate([pno, nsqo, pz], axis=1)],
        axis=0).astype(jnp.bfloat16)

    m = jax.lax.dot_general(
        xaug_ref[...], paug, (((1,), (1,)), ((), ())),
        preferred_element_type=jnp.float32)

    s_ref[...] += jnp.sum(jnp.exp(m), axis=1, keepdims=True)

    t = t_ref[...]
    ci = jax.lax.broadcasted_iota(jnp.int32, (1, PB), 1)
    pid = j * PB + jnp.where(ci < hp, 2 * ci, 2 * (ci - hp) + 1)
    mt_ref[...] += jnp.sum(jnp.where(pid == t, m, 0.0), axis=1,
                           keepdims=True)

    @pl.when(j == NSTEP - 1)
    def _():
        out_ref[0, 0] = jnp.sum(jnp.log(s_ref[...]) - mt_ref[...]) * (
            1.0 / BATCH)


@functools.partial(jax.jit, static_argnames=())
def kernel(X, indices, T, proxies):
    del indices
    tp = jnp.concatenate([T[0::2], T[1::2]]).reshape(BATCH, 1)
    out = pl.pallas_call(
        _loss_kernel,
        grid=(NSTEP,),
        in_specs=[
            pl.BlockSpec((BATCH // 2, 128), lambda j: (0, 0)),
            pl.BlockSpec((BATCH, 1), lambda j: (0, 0)),
            pl.BlockSpec((PB // 2, 128), lambda j: (j, 0)),
        ],
        out_specs=pl.BlockSpec((1, 1), lambda j: (0, 0),
                               memory_space=pltpu.SMEM),
        out_shape=jax.ShapeDtypeStruct((1, 1), jnp.float32),
        scratch_shapes=[
            pltpu.VMEM((BATCH, KAUG), jnp.bfloat16),
            pltpu.VMEM((BATCH, 1), jnp.float32),
            pltpu.VMEM((BATCH, 1), jnp.float32),
        ],
        compiler_params=pltpu.CompilerParams(
            dimension_semantics=("arbitrary",)),
    )(X.reshape(BATCH // 2, 128), tp, proxies.reshape(NB // 2, 128))
    return jnp.reshape(out, ())


# speedup vs baseline: 1.1144x; 1.0156x over previous
import functools

import jax
import jax.numpy as jnp
from jax.experimental import pallas as pl
from jax.experimental.pallas import tpu as pltpu

NB = 8192
EMB = 64
KAUG = 128
BATCH = 1024
PB = 2048
NSTEP = NB // PB
SCALE = 3.0


def _aug(V, width):
    sq = jnp.sum(V * V, axis=1, keepdims=True)
    Vn = V * (width / jnp.sqrt(sq + 1e-12))
    nsq = -(SCALE * SCALE) * sq / (sq + 1e-12)
    n = V.shape[0]
    return jnp.concatenate(
        [Vn, nsq, jnp.zeros((n, KAUG - EMB - 1), jnp.float32)],
        axis=1).astype(jnp.bfloat16)


def _loss_kernel(x_ref, t_ref, p_ref, out_ref, xaug_ref, tc_ref, s_ref,
                 mt_ref):
    j = pl.program_id(0)

    @pl.when(j == 0)
    def _():
        t8f = t_ref[...].astype(jnp.float32)
        rowi8 = jax.lax.broadcasted_iota(jnp.int32, (BATCH, 8), 0)
        g8 = jax.lax.broadcasted_iota(jnp.int32, (BATCH, 8), 1)
        R = ((rowi8 >> 7) == g8).astype(jnp.float32)
        S1 = jax.lax.dot_general(
            R, t8f, (((1,), (0,)), ((), ())),
            preferred_element_type=jnp.float32)
        rowi = jax.lax.broadcasted_iota(jnp.int32, (BATCH, 128), 0)
        ki = jax.lax.broadcasted_iota(jnp.int32, (BATCH, 128), 1)
        tcol = jnp.sum(jnp.where((rowi & 127) == ki, S1, 0.0), axis=1,
                       keepdims=True)
        tc_ref[...] = tcol.astype(jnp.int32)

        X2 = x_ref[...]
        sqxe = jnp.sum(X2[:, :EMB] * X2[:, :EMB], axis=1, keepdims=True)
        sqxo = jnp.sum(X2[:, EMB:] * X2[:, EMB:], axis=1, keepdims=True)
        x2e = X2[:, :EMB] * ((2.0 * SCALE) / jnp.sqrt(sqxe + 1e-12))
        x2o = X2[:, EMB:] * ((2.0 * SCALE) / jnp.sqrt(sqxo + 1e-12))
        h = BATCH // 2
        ones = jnp.ones((h, 1), jnp.float32)
        zeros = jnp.zeros((h, KAUG - EMB - 1), jnp.float32)
        auge = jnp.concatenate([x2e, ones, zeros], axis=1)
        augo = jnp.concatenate([x2o, ones, zeros], axis=1)
        xaug_ref[...] = jnp.concatenate(
            [auge, augo], axis=0).astype(jnp.bfloat16)
        s_ref[...] = jnp.zeros_like(s_ref)
        mt_ref[...] = jnp.zeros_like(mt_ref)

    P2 = p_ref[...]
    sqpe = jnp.sum(P2[:, :EMB] * P2[:, :EMB], axis=1, keepdims=True)
    sqpo = jnp.sum(P2[:, EMB:] * P2[:, EMB:], axis=1, keepdims=True)
    hp = PB // 2
    pne = P2[:, :EMB] * (SCALE / jnp.sqrt(sqpe + 1e-12))
    pno = P2[:, EMB:] * (SCALE / jnp.sqrt(sqpo + 1e-12))
    nsqe = -(SCALE * SCALE) * sqpe / (sqpe + 1e-12)
    nsqo = -(SCALE * SCALE) * sqpo / (sqpo + 1e-12)
    pz = jnp.zeros((hp, KAUG - EMB - 1), jnp.float32)
    paug = jnp.concatenate(
        [jnp.concatenate([pne, nsqe, pz], axis=1),
         jnp.concatenate([pno, nsqo, pz], axis=1)],
        axis=0).astype(jnp.bfloat16)

    m = jax.lax.dot_general(
        xaug_ref[...], paug, (((1,), (1,)), ((), ())),
        preferred_element_type=jnp.float32)

    s_ref[...] += jnp.sum(jnp.exp(m), axis=1, keepdims=True)

    t = tc_ref[...]
    ci = jax.lax.broadcasted_iota(jnp.int32, (1, PB), 1)
    pid = j * PB + jnp.where(ci < hp, 2 * ci, 2 * (ci - hp) + 1)
    mt_ref[...] += jnp.sum(jnp.where(pid == t, m, 0.0), axis=1,
                           keepdims=True)

    @pl.when(j == NSTEP - 1)
    def _():
        out_ref[0, 0] = jnp.sum(jnp.log(s_ref[...]) - mt_ref[...]) * (
            1.0 / BATCH)


@functools.partial(jax.jit, static_argnames=())
def kernel(X, indices, T, proxies):
    del indices
    tp = jnp.concatenate([T[0::2], T[1::2]]).reshape(8, 128)
    out = pl.pallas_call(
        _loss_kernel,
        grid=(NSTEP,),
        in_specs=[
            pl.BlockSpec((BATCH // 2, 128), lambda j: (0, 0)),
            pl.BlockSpec((8, 128), lambda j: (0, 0)),
            pl.BlockSpec((PB // 2, 128), lambda j: (j, 0)),
        ],
        out_specs=pl.BlockSpec((1, 1), lambda j: (0, 0),
                               memory_space=pltpu.SMEM),
        out_shape=jax.ShapeDtypeStruct((1, 1), jnp.float32),
        scratch_shapes=[
            pltpu.VMEM((BATCH, KAUG), jnp.bfloat16),
            pltpu.VMEM((BATCH, 1), jnp.int32),
            pltpu.VMEM((BATCH, 1), jnp.float32),
            pltpu.VMEM((BATCH, 1), jnp.float32),
        ],
        compiler_params=pltpu.CompilerParams(
            dimension_semantics=("arbitrary",)),
    )(X.reshape(BATCH // 2, 128), tp, proxies.reshape(NB // 2, 128))
    return jnp.reshape(out, ())
